# baseline (speedup 1.0000x reference)
import functools

import jax
import jax.numpy as jnp
import numpy as np
from jax import lax
from jax.experimental import pallas as pl
from jax.experimental.pallas import tpu as pltpu
from jax.experimental.pallas import tpu_sc as plsc

N = 10000
E = 320000
NFEAT = 128
D = 128
K = 8
DD = 16
NCLASS = 64

NC = 2
NS = 16
NW = NC * NS
EPW = E // NW
CH = 80
NCHUNK = EPW // CH
NK = N * K

_f32 = jnp.float32
_i32 = jnp.int32

_CAPSUM = np.zeros((D, D), np.float32)
for _k in range(K):
    _CAPSUM[_k * DD:(_k + 1) * DD, _k * DD:(_k + 1) * DD] = 1.0
_EXPAND8 = np.zeros((K, D), np.float32)
for _k in range(K):
    _EXPAND8[_k, _k * DD:(_k + 1) * DD] = 1.0

RB = 1000


def _prep_body(xnb_ref, w_ref, b_ref, cap_ref, o_ref):
    h = jnp.maximum(
        jax.lax.dot_general(xnb_ref[...], w_ref[...], (((1,), (0,)), ((), ())),
                            precision=lax.Precision.HIGHEST,
                            preferred_element_type=_f32) + b_ref[...], 0.0)
    q = jax.lax.dot_general(h * h, cap_ref[...], (((1,), (0,)), ((), ())),
                            precision=lax.Precision.HIGHEST,
                            preferred_element_type=_f32)
    o_ref[...] = h / jnp.maximum(jnp.sqrt(q), 1e-12)


def _prep(x_nb, W_pca, b_pca, cap):
    return pl.pallas_call(
        _prep_body,
        grid=(N // RB,),
        in_specs=[
            pl.BlockSpec((RB, NFEAT), lambda i: (i, 0)),
            pl.BlockSpec((NFEAT, D), lambda i: (0, 0)),
            pl.BlockSpec((1, D), lambda i: (0, 0)),
            pl.BlockSpec((D, D), lambda i: (0, 0)),
        ],
        out_specs=pl.BlockSpec((RB, D), lambda i: (i, 0)),
        out_shape=jax.ShapeDtypeStruct((N, D), _f32),
    )(x_nb, W_pca, b_pca, cap)


def _combine(i, v_ref, bnd_ref, bndr_ref):
    rows = i * RB + jax.lax.broadcasted_iota(_i32, (RB, NW), 0)
    oh = (rows == bndr_ref[...][:, 0][None, :]).astype(_f32)
    bndm = bnd_ref[...]
    vb = v_ref[...]
    feat = vb[:, :D] + jax.lax.dot_general(
        oh, bndm[:, :D], (((1,), (0,)), ((), ())),
        precision=lax.Precision.HIGHEST, preferred_element_type=_f32)
    s8 = vb[:, D:D + K] + jax.lax.dot_general(
        oh, bndm[:, D:D + K], (((1,), (0,)), ((), ())),
        precision=lax.Precision.HIGHEST, preferred_element_type=_f32)
    return feat, s8


def _fixup_body(norm, v_ref, bnd_ref, bndr_ref, exp_ref, cap_ref,
                u_ref, s_ref):
    i = pl.program_id(0)
    feat, s8 = _combine(i, v_ref, bnd_ref, bndr_ref)
    sfull = jax.lax.dot_general(s8, exp_ref[...], (((1,), (0,)), ((), ())),
                                precision=lax.Precision.HIGHEST,
                                preferred_element_type=_f32)
    u = jnp.where(sfull > 0.0, feat / jnp.maximum(sfull, 1e-30), 0.0)
    if norm:
        q = jax.lax.dot_general(u * u, cap_ref[...], (((1,), (0,)), ((), ())),
                                precision=lax.Precision.HIGHEST,
                                preferred_element_type=_f32)
        u = u / jnp.maximum(jnp.sqrt(q), 1e-12)
    u_ref[...] = u
    s_ref[...] = s8


def _fixup(v, bnd, bndr, exp8, cap, norm):
    return pl.pallas_call(
        functools.partial(_fixup_body, norm),
        grid=(N // RB,),
        in_specs=[
            pl.BlockSpec((RB, VW), lambda i: (i, 0)),
            pl.BlockSpec((NW, D + 16), lambda i: (0, 0)),
            pl.BlockSpec((NW, 16), lambda i: (0, 0)),
            pl.BlockSpec((K, D), lambda i: (0, 0)),
            pl.BlockSpec((D, D), lambda i: (0, 0)),
        ],
        out_specs=[
            pl.BlockSpec((RB, D), lambda i: (i, 0)),
            pl.BlockSpec((RB, K), lambda i: (i, 0)),
        ],
        out_shape=[
            jax.ShapeDtypeStruct((N, D), _f32),
            jax.ShapeDtypeStruct((N, K), _f32),
        ],
    )(v, bnd, bndr, exp8, cap)


def _final_body(v_ref, bnd_ref, bndr_ref, exp_ref, wm_ref, bm_ref,
                o_ref):
    i = pl.program_id(0)
    feat, s8 = _combine(i, v_ref, bnd_ref, bndr_ref)
    sfull = jax.lax.dot_general(s8, exp_ref[...], (((1,), (0,)), ((), ())),
                                precision=lax.Precision.HIGHEST,
                                preferred_element_type=_f32)
    u = jnp.where(sfull > 0.0, feat / jnp.maximum(sfull, 1e-30), 0.0)
    h = jnp.maximum(u, 0.0)
    logits = jax.lax.dot_general(h, wm_ref[...], (((1,), (0,)), ((), ())),
                                 precision=lax.Precision.HIGHEST,
                                 preferred_element_type=_f32) + bm_ref[...]
    m = jnp.max(logits, axis=1, keepdims=True)
    lse = jnp.log(jnp.sum(jnp.exp(logits - m), axis=1, keepdims=True))
    o_ref[...] = logits - m - lse


def _final(v, bnd, bndr, exp8, W_mlp, b_mlp):
    return pl.pallas_call(
        _final_body,
        grid=(N // RB,),
        in_specs=[
            pl.BlockSpec((RB, VW), lambda i: (i, 0)),
            pl.BlockSpec((NW, D + 16), lambda i: (0, 0)),
            pl.BlockSpec((NW, 16), lambda i: (0, 0)),
            pl.BlockSpec((K, D), lambda i: (0, 0)),
            pl.BlockSpec((D, NCLASS), lambda i: (0, 0)),
            pl.BlockSpec((1, NCLASS), lambda i: (0, 0)),
        ],
        out_specs=pl.BlockSpec((RB, NCLASS), lambda i: (i, 0)),
        out_shape=jax.ShapeDtypeStruct((N, NCLASS), _f32),
    )(v, bnd, bndr, exp8, W_mlp, b_mlp)


_MESH = plsc.VectorSubcoreMesh(core_axis_name="c", subcore_axis_name="s")
_SC_PARAMS = pltpu.CompilerParams(needs_layout_passes=False)

FB = 80
VW = 256
VROWS = N + 16
DUMP = N


def _sc_pass_body(has_u, x_hbm, u_hbm, ppr_hbm, row_hbm, col_hbm,
                  vout_hbm, bnd_hbm, bndr_hbm,
                  row_v, col_v, ppr_v, xc_v, ur_v, fb, fidx,
                  tmp_v, bnd_v, gsem, usem, isem):
    c = lax.axis_index("c")
    sid = lax.axis_index("s")
    wid = sid * NC + c
    iota = lax.iota(_i32, 16)
    z16 = jnp.zeros((16,), _f32)

    pltpu.sync_copy(row_hbm.at[pl.ds(wid * EPW, 16)], tmp_v)
    r_first = tmp_v[...][0]

    @pl.when(wid < NW - 1)
    def _probe():
        pltpu.sync_copy(row_hbm.at[pl.ds((wid + 1) * EPW, 16)], tmp_v)

    nx = tmp_v[...][0]
    r_next = jnp.where(wid < NW - 1, nx, -1)

    acc0 = tuple(z16 for _ in range(K)) + (z16,)

    def chunk(i, carry):
        base = wid * EPW + i * CH
        pltpu.sync_copy(row_hbm.at[pl.ds(base, CH)], row_v)
        pltpu.sync_copy(col_hbm.at[pl.ds(base, CH)], col_v)
        pltpu.sync_copy(ppr_hbm.at[pl.ds(base, CH)], ppr_v)
        cx = pltpu.async_copy(x_hbm.at[col_v], xc_v, gsem)
        if has_u:
            cu = pltpu.async_copy(u_hbm.at[row_v], ur_v, usem)
        cx.wait()
        if has_u:
            cu.wait()

        def group(g, gcarry):
            r_cur, nf, acc = gcarry
            pv = ppr_v[pl.ds(g * 16, 16)]
            rv = row_v[pl.ds(g * 16, 16)]
            if not has_u:
                wg = jnp.exp(pv)
            for l in range(16):
                e = g * 16 + l
                r = rv[l]
                xcs = [xc_v[e, pl.ds(j * DD, 16)] for j in range(K)]
                if has_u:
                    pe = pv[l]
                    tvec = jnp.zeros((16,), _f32)
                    for j in range(K):
                        tj = jnp.sum(xcs[j] * ur_v[e, pl.ds(j * DD, 16)])
                        tvec = jnp.where(iota == j, tj, tvec)
                    wv = jnp.where(iota < K, jnp.exp(pe * tvec), 0.0)
                else:
                    wv = jnp.where(iota < K, jnp.full((16,), wg[l], _f32),
                                   0.0)
                flush = r != r_cur

                @pl.when(flush)
                def _flush(nf=nf, r_cur=r_cur, acc=acc):
                    for j in range(K):
                        fb[nf, pl.ds(j * DD, 16)] = acc[j]
                    fb[nf, pl.ds(D, 16)] = acc[K]
                    plsc.store_scatter(fidx, [jnp.full((16,), nf, _i32)],
                                       jnp.full((16,), r_cur, _i32),
                                       mask=iota == 0)

                nf = nf + flush.astype(_i32)

                @pl.when(nf == FB)
                def _scat():
                    pltpu.sync_copy(fb, vout_hbm.at[fidx])

                nf = jnp.where(nf == FB, 0, nf)
                keep = jnp.where(flush, 0.0, 1.0)
                if has_u:
                    wjs = [jnp.full((16,), wv[j], _f32) for j in range(K)]
                else:
                    wjs = [wv] * K
                acc = tuple(acc[j] * keep + xcs[j] * wjs[j]
                            for j in range(K)) + (acc[K] * keep + wv,)
                r_cur = r
            return (r_cur, nf, acc)

        return lax.fori_loop(0, CH // 16, group, gcarry_init(carry))

    def gcarry_init(carry):
        return carry

    r_cur, nf, acc = lax.fori_loop(0, NCHUNK, chunk,
                                   (r_first, jnp.int32(0), acc0))

    for j in range(K):
        bnd_v[0, pl.ds(j * DD, 16)] = acc[j]
    bnd_v[0, pl.ds(D, 16)] = acc[K]
    cont = r_cur == r_next
    bndrow = jnp.where(cont, r_cur, DUMP)
    tmp_v[...] = jnp.full((16,), bndrow, _i32)
    pltpu.sync_copy(bnd_v, bnd_hbm.at[wid])
    pltpu.sync_copy(tmp_v, bndr_hbm.at[wid])

    @pl.when(jnp.logical_not(cont))
    def _last(nf=nf):
        for j in range(K):
            fb[nf, pl.ds(j * DD, 16)] = acc[j]
        fb[nf, pl.ds(D, 16)] = acc[K]
        plsc.store_scatter(fidx, [jnp.full((16,), nf, _i32)],
                           jnp.full((16,), r_cur, _i32), mask=iota == 0)

    nf = nf + jnp.logical_not(cont).astype(_i32)

    def pad(i, carry):
        @pl.when(i >= nf)
        def _():
            plsc.store_scatter(fidx, [jnp.full((16,), i, _i32)],
                               jnp.full((16,), DUMP, _i32), mask=iota == 0)
        return carry

    lax.fori_loop(0, FB, pad, 0)
    pltpu.sync_copy(fb, vout_hbm.at[fidx])


_SC_OUT = (
    jax.ShapeDtypeStruct((VROWS, VW), _f32),
    jax.ShapeDtypeStruct((NW, 1, D + 16), _f32),
    jax.ShapeDtypeStruct((NW, 16), _i32),
)

_SC_SCRATCH = [
    pltpu.VMEM((CH,), _i32),
    pltpu.VMEM((CH,), _i32),
    pltpu.VMEM((CH,), _f32),
    pltpu.VMEM((CH, D), _f32),
    pltpu.VMEM((CH, D), _f32),
    pltpu.VMEM((FB, VW), _f32),
    pltpu.VMEM((FB,), _i32),
    pltpu.VMEM((16,), _i32),
    pltpu.VMEM((1, D + 16), _f32),
    pltpu.SemaphoreType.DMA,
    pltpu.SemaphoreType.DMA,
    pltpu.SemaphoreType.DMA,
]


@functools.partial(pl.kernel, out_type=_SC_OUT, mesh=_MESH,
                   compiler_params=_SC_PARAMS, scratch_types=_SC_SCRATCH)
def _sc_iter(x_hbm, u_hbm, ppr_hbm, row_hbm, col_hbm,
             vout_hbm, bnd_hbm, bndr_hbm,
             row_v, col_v, ppr_v, xc_v, ur_v, fb, fidx,
             tmp_v, bnd_v, gsem, usem, isem):
    _sc_pass_body(True, x_hbm, u_hbm, ppr_hbm, row_hbm, col_hbm,
                  vout_hbm, bnd_hbm, bndr_hbm,
                  row_v, col_v, ppr_v, xc_v, ur_v, fb, fidx,
                  tmp_v, bnd_v, gsem, usem, isem)


@functools.partial(pl.kernel, out_type=_SC_OUT, mesh=_MESH,
                   compiler_params=_SC_PARAMS, scratch_types=_SC_SCRATCH)
def _sc_init(x_hbm, u_hbm, ppr_hbm, row_hbm, col_hbm,
             vout_hbm, bnd_hbm, bndr_hbm,
             row_v, col_v, ppr_v, xc_v, ur_v, fb, fidx,
             tmp_v, bnd_v, gsem, usem, isem):
    _sc_pass_body(False, x_hbm, u_hbm, ppr_hbm, row_hbm, col_hbm,
                  vout_hbm, bnd_hbm, bndr_hbm,
                  row_v, col_v, ppr_v, xc_v, ur_v, fb, fidx,
                  tmp_v, bnd_v, gsem, usem, isem)


@functools.partial(
    pl.kernel,
    out_type=jax.ShapeDtypeStruct((E,), _f32),
    mesh=_MESH,
    compiler_params=_SC_PARAMS,
    scratch_types=[
        pltpu.VMEM((N,), _f32),
        pltpu.VMEM((CH,), _i32),
        pltpu.VMEM((CH,), _f32),
        pltpu.VMEM((CH,), _f32),
    ],
)
def _sc_pprs(ppr_hbm, row_hbm, s0_hbm, out_hbm, s0_v, row_v, ppr_v, o_v):
    wid = lax.axis_index("s") * NC + lax.axis_index("c")
    pltpu.sync_copy(s0_hbm, s0_v)

    def chunk(i, carry):
        base = wid * EPW + i * CH
        pltpu.sync_copy(row_hbm.at[pl.ds(base, CH)], row_v)
        pltpu.sync_copy(ppr_hbm.at[pl.ds(base, CH)], ppr_v)
        for g in range(CH // 16):
            rid = row_v[pl.ds(g * 16, 16)]
            sv = plsc.load_gather(s0_v, [rid])
            o_v[pl.ds(g * 16, 16)] = (
                jnp.exp(ppr_v[pl.ds(g * 16, 16)]) / jnp.maximum(sv, 1e-30))
        pltpu.sync_copy(o_v, out_hbm.at[pl.ds(base, CH)])
        return carry

    lax.fori_loop(0, NCHUNK, chunk, 0)


def kernel(x_nb, ppr, row_idx, col_idx, x_idx, W_pca, b_pca, W_mlp, b_mlp):
    del x_idx
    row_idx = row_idx.astype(_i32)
    col_idx = col_idx.astype(_i32)
    cap = jnp.asarray(_CAPSUM)
    exp8 = jnp.asarray(_EXPAND8)

    x = _prep(x_nb, W_pca, b_pca.reshape(1, D), cap)
    v, bnd, bndr = _sc_init(x, x, ppr, row_idx, col_idx)
    u, s0 = _fixup(v, bnd.reshape(NW, D + 16), bndr, exp8, cap, norm=False)
    pprs = _sc_pprs(ppr, row_idx, s0[:, 0])
    for it in range(3):
        v, bnd, bndr = _sc_iter(x, u, pprs, row_idx, col_idx)
        if it < 2:
            u, _ = _fixup(v, bnd.reshape(NW, D + 16), bndr, exp8, cap,
                          norm=True)
    return _final(v, bnd.reshape(NW, D + 16), bndr, exp8, W_mlp,
                  b_mlp.reshape(1, NCLASS))


# speedup vs baseline: 1.5846x; 1.5846x over previous
import functools

import jax
import jax.numpy as jnp
import numpy as np
from jax import lax
from jax.experimental import pallas as pl
from jax.experimental.pallas import tpu as pltpu
from jax.experimental.pallas import tpu_sc as plsc

N = 10000
E = 320000
NFEAT = 128
D = 128
K = 8
DD = 16
NCLASS = 64

NC = 2
NS = 16
NW = NC * NS
EPW = E // NW
CH = 80
NCHUNK = EPW // CH
NK = N * K

_f32 = jnp.float32
_i32 = jnp.int32

_CAPSUM = np.zeros((D, D), np.float32)
for _k in range(K):
    _CAPSUM[_k * DD:(_k + 1) * DD, _k * DD:(_k + 1) * DD] = 1.0
_EXPAND8 = np.zeros((K, D), np.float32)
for _k in range(K):
    _EXPAND8[_k, _k * DD:(_k + 1) * DD] = 1.0

RB = 1000


def _prep_body(xnb_ref, w_ref, b_ref, cap_ref, o_ref):
    h = jnp.maximum(
        jax.lax.dot_general(xnb_ref[...], w_ref[...], (((1,), (0,)), ((), ())),
                            precision=lax.Precision.HIGHEST,
                            preferred_element_type=_f32) + b_ref[...], 0.0)
    q = jax.lax.dot_general(h * h, cap_ref[...], (((1,), (0,)), ((), ())),
                            precision=lax.Precision.HIGHEST,
                            preferred_element_type=_f32)
    o_ref[...] = h / jnp.maximum(jnp.sqrt(q), 1e-12)


def _prep(x_nb, W_pca, b_pca, cap):
    return pl.pallas_call(
        _prep_body,
        grid=(N // RB,),
        in_specs=[
            pl.BlockSpec((RB, NFEAT), lambda i: (i, 0)),
            pl.BlockSpec((NFEAT, D), lambda i: (0, 0)),
            pl.BlockSpec((1, D), lambda i: (0, 0)),
            pl.BlockSpec((D, D), lambda i: (0, 0)),
        ],
        out_specs=pl.BlockSpec((RB, D), lambda i: (i, 0)),
        out_shape=jax.ShapeDtypeStruct((N, D), _f32),
    )(x_nb, W_pca, b_pca, cap)


def _combine(i, v_ref, bnd_ref, bndr_ref):
    rows = i * RB + jax.lax.broadcasted_iota(_i32, (RB, NW), 0)
    oh = (rows == bndr_ref[...][:, 0][None, :]).astype(_f32)
    return v_ref[...] + jax.lax.dot_general(
        oh, bnd_ref[...], (((1,), (0,)), ((), ())),
        precision=lax.Precision.HIGHEST, preferred_element_type=_f32)


def _fixup_body(norm, v_ref, sp_ref, bnd_ref, bndr_ref, exp_ref, cap_ref,
                u_ref, s_ref):
    i = pl.program_id(0)
    feat = _combine(i, v_ref, bnd_ref, bndr_ref)
    s8 = jnp.sum(sp_ref[...], axis=0)
    sfull = jax.lax.dot_general(s8, exp_ref[...], (((1,), (0,)), ((), ())),
                                precision=lax.Precision.HIGHEST,
                                preferred_element_type=_f32)
    u = jnp.where(sfull > 0.0, feat / jnp.maximum(sfull, 1e-30), 0.0)
    if norm:
        q = jax.lax.dot_general(u * u, cap_ref[...], (((1,), (0,)), ((), ())),
                                precision=lax.Precision.HIGHEST,
                                preferred_element_type=_f32)
        u = u / jnp.maximum(jnp.sqrt(q), 1e-12)
    u_ref[...] = u
    s_ref[...] = s8


def _fixup(v, sp, bnd, bndr, exp8, cap, norm):
    return pl.pallas_call(
        functools.partial(_fixup_body, norm),
        grid=(N // RB,),
        in_specs=[
            pl.BlockSpec((RB, D), lambda i: (i, 0)),
            pl.BlockSpec((NW, RB, K), lambda i: (0, i, 0)),
            pl.BlockSpec((NW, D), lambda i: (0, 0)),
            pl.BlockSpec((NW, 16), lambda i: (0, 0)),
            pl.BlockSpec((K, D), lambda i: (0, 0)),
            pl.BlockSpec((D, D), lambda i: (0, 0)),
        ],
        out_specs=[
            pl.BlockSpec((RB, D), lambda i: (i, 0)),
            pl.BlockSpec((RB, K), lambda i: (i, 0)),
        ],
        out_shape=[
            jax.ShapeDtypeStruct((N, D), _f32),
            jax.ShapeDtypeStruct((N, K), _f32),
        ],
    )(v, sp, bnd, bndr, exp8, cap)


def _final_body(v_ref, sp_ref, bnd_ref, bndr_ref, exp_ref, wm_ref, bm_ref,
                o_ref):
    i = pl.program_id(0)
    feat = _combine(i, v_ref, bnd_ref, bndr_ref)
    s8 = jnp.sum(sp_ref[...], axis=0)
    sfull = jax.lax.dot_general(s8, exp_ref[...], (((1,), (0,)), ((), ())),
                                precision=lax.Precision.HIGHEST,
                                preferred_element_type=_f32)
    u = jnp.where(sfull > 0.0, feat / jnp.maximum(sfull, 1e-30), 0.0)
    h = jnp.maximum(u, 0.0)
    logits = jax.lax.dot_general(h, wm_ref[...], (((1,), (0,)), ((), ())),
                                 precision=lax.Precision.HIGHEST,
                                 preferred_element_type=_f32) + bm_ref[...]
    m = jnp.max(logits, axis=1, keepdims=True)
    lse = jnp.log(jnp.sum(jnp.exp(logits - m), axis=1, keepdims=True))
    o_ref[...] = logits - m - lse


def _final(v, sp, bnd, bndr, exp8, W_mlp, b_mlp):
    return pl.pallas_call(
        _final_body,
        grid=(N // RB,),
        in_specs=[
            pl.BlockSpec((RB, D), lambda i: (i, 0)),
            pl.BlockSpec((NW, RB, K), lambda i: (0, i, 0)),
            pl.BlockSpec((NW, D), lambda i: (0, 0)),
            pl.BlockSpec((NW, 16), lambda i: (0, 0)),
            pl.BlockSpec((K, D), lambda i: (0, 0)),
            pl.BlockSpec((D, NCLASS), lambda i: (0, 0)),
            pl.BlockSpec((1, NCLASS), lambda i: (0, 0)),
        ],
        out_specs=pl.BlockSpec((RB, NCLASS), lambda i: (i, 0)),
        out_shape=jax.ShapeDtypeStruct((N, NCLASS), _f32),
    )(v, sp, bnd, bndr, exp8, W_mlp, b_mlp)


_MESH = plsc.VectorSubcoreMesh(core_axis_name="c", subcore_axis_name="s")
_SC_PARAMS = pltpu.CompilerParams(needs_layout_passes=False)

FB = 80
VROWS = N + 16
DUMP = N


def _sc_pass_body(has_u, x_hbm, u_hbm, ppr_hbm, row_hbm, col_hbm,
                  vout_hbm, sout_hbm, bnd_hbm, bndr_hbm,
                  s_acc, row_v, col_v, ppr_v, xc_v, ur_v, fb, fidx,
                  tmp_v, bnd_v, gsem, usem):
    c = lax.axis_index("c")
    sid = lax.axis_index("s")
    wid = sid * NC + c
    iota = lax.iota(_i32, 16)
    z16 = jnp.zeros((16,), _f32)

    def zloop(i, carry):
        s_acc[pl.ds(i * 16, 16)] = z16
        return carry

    lax.fori_loop(0, NK // 16, zloop, 0)

    pltpu.sync_copy(row_hbm.at[pl.ds(wid * EPW, 16)], tmp_v)
    r_first = tmp_v[...][0]

    @pl.when(wid < NW - 1)
    def _probe():
        pltpu.sync_copy(row_hbm.at[pl.ds((wid + 1) * EPW, 16)], tmp_v)

    nx = tmp_v[...][0]
    r_next = jnp.where(wid < NW - 1, nx, -1)

    acc0 = tuple(z16 for _ in range(K))

    def chunk(i, carry):
        base = wid * EPW + i * CH
        pltpu.sync_copy(row_hbm.at[pl.ds(base, CH)], row_v)
        pltpu.sync_copy(col_hbm.at[pl.ds(base, CH)], col_v)
        pltpu.sync_copy(ppr_hbm.at[pl.ds(base, CH)], ppr_v)
        cx = pltpu.async_copy(x_hbm.at[col_v], xc_v, gsem)
        if has_u:
            cu = pltpu.async_copy(u_hbm.at[row_v], ur_v, usem)
        cx.wait()
        if has_u:
            cu.wait()

        def group(g, gcarry):
            r_cur, nf, acc = gcarry
            pv = ppr_v[pl.ds(g * 16, 16)]
            rv = row_v[pl.ds(g * 16, 16)]
            if not has_u:
                wg = jnp.exp(pv)
            for l in range(16):
                e = g * 16 + l
                r = rv[l]
                xcs = [xc_v[e, pl.ds(j * DD, 16)] for j in range(K)]
                if has_u:
                    pe = pv[l]
                    tvec = jnp.zeros((16,), _f32)
                    for j in range(K):
                        tj = jnp.sum(xcs[j] * ur_v[e, pl.ds(j * DD, 16)])
                        tvec = jnp.where(iota == j, tj, tvec)
                    wv = jnp.where(iota < K, jnp.exp(pe * tvec), 0.0)
                else:
                    wv = jnp.where(iota < K, jnp.full((16,), wg[l], _f32),
                                   0.0)
                plsc.addupdate_scatter(s_acc, [r * K + iota], wv,
                                       mask=iota < K)
                flush = r != r_cur

                @pl.when(flush)
                def _flush(nf=nf, r_cur=r_cur, acc=acc):
                    for j in range(K):
                        fb[nf, pl.ds(j * DD, 16)] = acc[j]
                    plsc.store_scatter(fidx, [jnp.full((16,), nf, _i32)],
                                       jnp.full((16,), r_cur, _i32),
                                       mask=iota == 0)

                nf = nf + flush.astype(_i32)

                @pl.when(nf == FB)
                def _scat():
                    pltpu.sync_copy(fb, vout_hbm.at[fidx])

                nf = jnp.where(nf == FB, 0, nf)
                keep = jnp.where(flush, 0.0, 1.0)
                if has_u:
                    wjs = [jnp.full((16,), wv[j], _f32) for j in range(K)]
                else:
                    wjs = [wv] * K
                acc = tuple(acc[j] * keep + xcs[j] * wjs[j]
                            for j in range(K))
                r_cur = r
            return (r_cur, nf, acc)

        return lax.fori_loop(0, CH // 16, group, carry)

    r_cur, nf, acc = lax.fori_loop(0, NCHUNK, chunk,
                                   (r_first, jnp.int32(0), acc0))

    for j in range(K):
        bnd_v[0, pl.ds(j * DD, 16)] = acc[j]
    cont = r_cur == r_next
    bndrow = jnp.where(cont, r_cur, DUMP)
    tmp_v[...] = jnp.full((16,), bndrow, _i32)
    pltpu.sync_copy(bnd_v, bnd_hbm.at[wid])
    pltpu.sync_copy(tmp_v, bndr_hbm.at[wid])

    @pl.when(jnp.logical_not(cont))
    def _last(nf=nf):
        for j in range(K):
            fb[nf, pl.ds(j * DD, 16)] = acc[j]
        plsc.store_scatter(fidx, [jnp.full((16,), nf, _i32)],
                           jnp.full((16,), r_cur, _i32), mask=iota == 0)

    nf = nf + jnp.logical_not(cont).astype(_i32)

    def pad(i, carry):
        @pl.when(i >= nf)
        def _():
            plsc.store_scatter(fidx, [jnp.full((16,), i, _i32)],
                               jnp.full((16,), DUMP, _i32), mask=iota == 0)
        return carry

    lax.fori_loop(0, FB, pad, 0)
    pltpu.sync_copy(fb, vout_hbm.at[fidx])
    pltpu.sync_copy(s_acc, sout_hbm.at[wid])


_SC_OUT = (
    jax.ShapeDtypeStruct((VROWS, D), _f32),
    jax.ShapeDtypeStruct((NW, NK), _f32),
    jax.ShapeDtypeStruct((NW, 1, D), _f32),
    jax.ShapeDtypeStruct((NW, 16), _i32),
)

_SC_SCRATCH = [
    pltpu.VMEM((NK,), _f32),
    pltpu.VMEM((CH,), _i32),
    pltpu.VMEM((CH,), _i32),
    pltpu.VMEM((CH,), _f32),
    pltpu.VMEM((CH, D), _f32),
    pltpu.VMEM((CH, D), _f32),
    pltpu.VMEM((FB, D), _f32),
    pltpu.VMEM((FB,), _i32),
    pltpu.VMEM((16,), _i32),
    pltpu.VMEM((1, D), _f32),
    pltpu.SemaphoreType.DMA,
    pltpu.SemaphoreType.DMA,
]


@functools.partial(pl.kernel, out_type=_SC_OUT, mesh=_MESH,
                   compiler_params=_SC_PARAMS, scratch_types=_SC_SCRATCH)
def _sc_iter(x_hbm, u_hbm, ppr_hbm, row_hbm, col_hbm,
             vout_hbm, sout_hbm, bnd_hbm, bndr_hbm,
             s_acc, row_v, col_v, ppr_v, xc_v, ur_v, fb, fidx,
             tmp_v, bnd_v, gsem, usem):
    _sc_pass_body(True, x_hbm, u_hbm, ppr_hbm, row_hbm, col_hbm,
                  vout_hbm, sout_hbm, bnd_hbm, bndr_hbm,
                  s_acc, row_v, col_v, ppr_v, xc_v, ur_v, fb, fidx,
                  tmp_v, bnd_v, gsem, usem)


@functools.partial(pl.kernel, out_type=_SC_OUT, mesh=_MESH,
                   compiler_params=_SC_PARAMS, scratch_types=_SC_SCRATCH)
def _sc_init(x_hbm, u_hbm, ppr_hbm, row_hbm, col_hbm,
             vout_hbm, sout_hbm, bnd_hbm, bndr_hbm,
             s_acc, row_v, col_v, ppr_v, xc_v, ur_v, fb, fidx,
             tmp_v, bnd_v, gsem, usem):
    _sc_pass_body(False, x_hbm, u_hbm, ppr_hbm, row_hbm, col_hbm,
                  vout_hbm, sout_hbm, bnd_hbm, bndr_hbm,
                  s_acc, row_v, col_v, ppr_v, xc_v, ur_v, fb, fidx,
                  tmp_v, bnd_v, gsem, usem)


@functools.partial(
    pl.kernel,
    out_type=jax.ShapeDtypeStruct((E,), _f32),
    mesh=_MESH,
    compiler_params=_SC_PARAMS,
    scratch_types=[
        pltpu.VMEM((N,), _f32),
        pltpu.VMEM((CH,), _i32),
        pltpu.VMEM((CH,), _f32),
        pltpu.VMEM((CH,), _f32),
    ],
)
def _sc_pprs(ppr_hbm, row_hbm, s0_hbm, out_hbm, s0_v, row_v, ppr_v, o_v):
    wid = lax.axis_index("s") * NC + lax.axis_index("c")
    pltpu.sync_copy(s0_hbm, s0_v)

    def chunk(i, carry):
        base = wid * EPW + i * CH
        pltpu.sync_copy(row_hbm.at[pl.ds(base, CH)], row_v)
        pltpu.sync_copy(ppr_hbm.at[pl.ds(base, CH)], ppr_v)
        for g in range(CH // 16):
            rid = row_v[pl.ds(g * 16, 16)]
            sv = plsc.load_gather(s0_v, [rid])
            o_v[pl.ds(g * 16, 16)] = (
                jnp.exp(ppr_v[pl.ds(g * 16, 16)]) / jnp.maximum(sv, 1e-30))
        pltpu.sync_copy(o_v, out_hbm.at[pl.ds(base, CH)])
        return carry

    lax.fori_loop(0, NCHUNK, chunk, 0)


def kernel(x_nb, ppr, row_idx, col_idx, x_idx, W_pca, b_pca, W_mlp, b_mlp):
    del x_idx
    row_idx = row_idx.astype(_i32)
    col_idx = col_idx.astype(_i32)
    cap = jnp.asarray(_CAPSUM)
    exp8 = jnp.asarray(_EXPAND8)

    x = _prep(x_nb, W_pca, b_pca.reshape(1, D), cap)
    v, sout, bnd, bndr = _sc_init(x, x, ppr, row_idx, col_idx)
    u, s0 = _fixup(v, sout.reshape(NW, N, K), bnd.reshape(NW, D), bndr,
                   exp8, cap, norm=False)
    pprs = _sc_pprs(ppr, row_idx, s0[:, 0])
    for it in range(3):
        v, sout, bnd, bndr = _sc_iter(x, u, pprs, row_idx, col_idx)
        if it < 2:
            u, _ = _fixup(v, sout.reshape(NW, N, K), bnd.reshape(NW, D),
                          bndr, exp8, cap, norm=True)
    return _final(v, sout.reshape(NW, N, K), bnd.reshape(NW, D), bndr,
                  exp8, W_mlp, b_mlp.reshape(1, NCLASS))


# speedup vs baseline: 1.6370x; 1.0331x over previous
import functools

import jax
import jax.numpy as jnp
import numpy as np
from jax import lax
from jax.experimental import pallas as pl
from jax.experimental.pallas import tpu as pltpu
from jax.experimental.pallas import tpu_sc as plsc

N = 10000
E = 320000
NFEAT = 128
D = 128
K = 8
DD = 16
NCLASS = 64

NC = 2
NS = 16
NW = NC * NS
EPW = E // NW
CH = 80
NCHUNK = EPW // CH
NK = N * K

_f32 = jnp.float32
_i32 = jnp.int32

_CAPSUM = np.zeros((D, D), np.float32)
for _k in range(K):
    _CAPSUM[_k * DD:(_k + 1) * DD, _k * DD:(_k + 1) * DD] = 1.0
_EXPAND8 = np.zeros((K, D), np.float32)
for _k in range(K):
    _EXPAND8[_k, _k * DD:(_k + 1) * DD] = 1.0

RB = 1000


def _prep_body(xnb_ref, w_ref, b_ref, cap_ref, o_ref):
    h = jnp.maximum(
        jax.lax.dot_general(xnb_ref[...], w_ref[...], (((1,), (0,)), ((), ())),
                            precision=lax.Precision.HIGHEST,
                            preferred_element_type=_f32) + b_ref[...], 0.0)
    q = jax.lax.dot_general(h * h, cap_ref[...], (((1,), (0,)), ((), ())),
                            precision=lax.Precision.HIGHEST,
                            preferred_element_type=_f32)
    o_ref[...] = h / jnp.maximum(jnp.sqrt(q), 1e-12)


def _prep(x_nb, W_pca, b_pca, cap):
    return pl.pallas_call(
        _prep_body,
        grid=(N // RB,),
        in_specs=[
            pl.BlockSpec((RB, NFEAT), lambda i: (i, 0)),
            pl.BlockSpec((NFEAT, D), lambda i: (0, 0)),
            pl.BlockSpec((1, D), lambda i: (0, 0)),
            pl.BlockSpec((D, D), lambda i: (0, 0)),
        ],
        out_specs=pl.BlockSpec((RB, D), lambda i: (i, 0)),
        out_shape=jax.ShapeDtypeStruct((N, D), _f32),
    )(x_nb, W_pca, b_pca, cap)


def _combine(i, v_ref, bnd_ref, bndr_ref):
    rows = i * RB + jax.lax.broadcasted_iota(_i32, (RB, NW), 0)
    oh = (rows == bndr_ref[...][:, 0][None, :]).astype(_f32)
    return v_ref[...] + jax.lax.dot_general(
        oh, bnd_ref[...], (((1,), (0,)), ((), ())),
        precision=lax.Precision.HIGHEST, preferred_element_type=_f32)


def _fixup_body(norm, v_ref, sp_ref, bnd_ref, bndr_ref, exp_ref, cap_ref,
                u_ref, s_ref):
    i = pl.program_id(0)
    feat = _combine(i, v_ref, bnd_ref, bndr_ref)
    s8 = jnp.sum(sp_ref[...], axis=0)
    sfull = jax.lax.dot_general(s8, exp_ref[...], (((1,), (0,)), ((), ())),
                                precision=lax.Precision.HIGHEST,
                                preferred_element_type=_f32)
    u = jnp.where(sfull > 0.0, feat / jnp.maximum(sfull, 1e-30), 0.0)
    if norm:
        q = jax.lax.dot_general(u * u, cap_ref[...], (((1,), (0,)), ((), ())),
                                precision=lax.Precision.HIGHEST,
                                preferred_element_type=_f32)
        u = u / jnp.maximum(jnp.sqrt(q), 1e-12)
    u_ref[...] = u
    s_ref[...] = s8


def _fixup(v, sp, bnd, bndr, exp8, cap, norm):
    return pl.pallas_call(
        functools.partial(_fixup_body, norm),
        grid=(N // RB,),
        in_specs=[
            pl.BlockSpec((RB, D), lambda i: (i, 0)),
            pl.BlockSpec((NW, RB, K), lambda i: (0, i, 0)),
            pl.BlockSpec((NW, D), lambda i: (0, 0)),
            pl.BlockSpec((NW, 16), lambda i: (0, 0)),
            pl.BlockSpec((K, D), lambda i: (0, 0)),
            pl.BlockSpec((D, D), lambda i: (0, 0)),
        ],
        out_specs=[
            pl.BlockSpec((RB, D), lambda i: (i, 0)),
            pl.BlockSpec((RB, K), lambda i: (i, 0)),
        ],
        out_shape=[
            jax.ShapeDtypeStruct((N, D), _f32),
            jax.ShapeDtypeStruct((N, K), _f32),
        ],
    )(v, sp, bnd, bndr, exp8, cap)


def _final_body(v_ref, sp_ref, bnd_ref, bndr_ref, exp_ref, wm_ref, bm_ref,
                o_ref):
    i = pl.program_id(0)
    feat = _combine(i, v_ref, bnd_ref, bndr_ref)
    s8 = jnp.sum(sp_ref[...], axis=0)
    sfull = jax.lax.dot_general(s8, exp_ref[...], (((1,), (0,)), ((), ())),
                                precision=lax.Precision.HIGHEST,
                                preferred_element_type=_f32)
    u = jnp.where(sfull > 0.0, feat / jnp.maximum(sfull, 1e-30), 0.0)
    h = jnp.maximum(u, 0.0)
    logits = jax.lax.dot_general(h, wm_ref[...], (((1,), (0,)), ((), ())),
                                 precision=lax.Precision.HIGHEST,
                                 preferred_element_type=_f32) + bm_ref[...]
    m = jnp.max(logits, axis=1, keepdims=True)
    lse = jnp.log(jnp.sum(jnp.exp(logits - m), axis=1, keepdims=True))
    o_ref[...] = logits - m - lse


def _final(v, sp, bnd, bndr, exp8, W_mlp, b_mlp):
    return pl.pallas_call(
        _final_body,
        grid=(N // RB,),
        in_specs=[
            pl.BlockSpec((RB, D), lambda i: (i, 0)),
            pl.BlockSpec((NW, RB, K), lambda i: (0, i, 0)),
            pl.BlockSpec((NW, D), lambda i: (0, 0)),
            pl.BlockSpec((NW, 16), lambda i: (0, 0)),
            pl.BlockSpec((K, D), lambda i: (0, 0)),
            pl.BlockSpec((D, NCLASS), lambda i: (0, 0)),
            pl.BlockSpec((1, NCLASS), lambda i: (0, 0)),
        ],
        out_specs=pl.BlockSpec((RB, NCLASS), lambda i: (i, 0)),
        out_shape=jax.ShapeDtypeStruct((N, NCLASS), _f32),
    )(v, sp, bnd, bndr, exp8, W_mlp, b_mlp)


_MESH = plsc.VectorSubcoreMesh(core_axis_name="c", subcore_axis_name="s")
_SC_PARAMS = pltpu.CompilerParams(needs_layout_passes=False)

FB = 80
VROWS = N + 16
DUMP = N


def _sc_pass_body(has_u, x_hbm, u_hbm, ppr_hbm, row_hbm, col_hbm,
                  vout_hbm, sout_hbm, bnd_hbm, bndr_hbm,
                  s_acc, row_v, col_v, ppr_v, xc_v, ur_v, fb, fidx,
                  tmp_v, bnd_v, gsem, usem):
    c = lax.axis_index("c")
    sid = lax.axis_index("s")
    wid = sid * NC + c
    iota = lax.iota(_i32, 16)
    z16 = jnp.zeros((16,), _f32)

    def zloop(i, carry):
        s_acc[pl.ds(i * 16, 16)] = z16
        return carry

    lax.fori_loop(0, NK // 16, zloop, 0)
    for g in range(FB // 16):
        fidx[pl.ds(g * 16, 16)] = jnp.full((16,), DUMP, _i32)

    pltpu.sync_copy(row_hbm.at[pl.ds(wid * EPW, 16)], tmp_v)
    r_first = tmp_v[...][0]

    @pl.when(wid < NW - 1)
    def _probe():
        pltpu.sync_copy(row_hbm.at[pl.ds((wid + 1) * EPW, 16)], tmp_v)

    nx = tmp_v[...][0]
    r_next = jnp.where(wid < NW - 1, nx, -1)

    acc0 = tuple(z16 for _ in range(K))

    def chunk(i, carry):
        base = wid * EPW + i * CH
        pltpu.sync_copy(row_hbm.at[pl.ds(base, CH)], row_v)
        pltpu.sync_copy(col_hbm.at[pl.ds(base, CH)], col_v)
        pltpu.sync_copy(ppr_hbm.at[pl.ds(base, CH)], ppr_v)
        cx = pltpu.async_copy(x_hbm.at[col_v], xc_v, gsem)
        if has_u:
            cu = pltpu.async_copy(u_hbm.at[row_v], ur_v, usem)
        cx.wait()
        if has_u:
            cu.wait()

        def group(g, gcarry):
            r_cur, nf, acc = gcarry

            @pl.when(nf + 16 > FB)
            def _scat():
                pltpu.sync_copy(fb, vout_hbm.at[fidx])

            nf = jnp.where(nf + 16 > FB, 0, nf)
            pv = ppr_v[pl.ds(g * 16, 16)]
            rv = row_v[pl.ds(g * 16, 16)]
            if not has_u:
                wg = jnp.exp(pv)
            for l in range(16):
                e = g * 16 + l
                r = rv[l]
                xcs = [xc_v[e, pl.ds(j * DD, 16)] for j in range(K)]
                if has_u:
                    pe = pv[l]
                    tvec = jnp.zeros((16,), _f32)
                    for j in range(K):
                        tj = jnp.sum(xcs[j] * ur_v[e, pl.ds(j * DD, 16)])
                        tvec = jnp.where(iota == j, tj, tvec)
                    wv = jnp.where(iota < K, jnp.exp(pe * tvec), 0.0)
                else:
                    wv = jnp.where(iota < K, jnp.full((16,), wg[l], _f32),
                                   0.0)
                plsc.addupdate_scatter(s_acc, [r * K + iota], wv,
                                       mask=iota < K)
                flush = r != r_cur

                @pl.when(flush)
                def _flush(nf=nf, r_cur=r_cur, acc=acc):
                    for j in range(K):
                        fb[nf, pl.ds(j * DD, 16)] = acc[j]
                    plsc.store_scatter(fidx, [jnp.full((16,), nf, _i32)],
                                       jnp.full((16,), r_cur, _i32),
                                       mask=iota == 0)

                nf = nf + flush.astype(_i32)
                keep = jnp.where(flush, 0.0, 1.0)
                if has_u:
                    wjs = [jnp.full((16,), wv[j], _f32) for j in range(K)]
                else:
                    wjs = [wv] * K
                acc = tuple(acc[j] * keep + xcs[j] * wjs[j]
                            for j in range(K))
                r_cur = r
            return (r_cur, nf, acc)

        return lax.fori_loop(0, CH // 16, group, carry)

    r_cur, nf, acc = lax.fori_loop(0, NCHUNK, chunk,
                                   (r_first, jnp.int32(0), acc0))

    for j in range(K):
        bnd_v[0, pl.ds(j * DD, 16)] = acc[j]
    cont = r_cur == r_next
    bndrow = jnp.where(cont, r_cur, DUMP)
    tmp_v[...] = jnp.full((16,), bndrow, _i32)
    pltpu.sync_copy(bnd_v, bnd_hbm.at[wid])
    pltpu.sync_copy(tmp_v, bndr_hbm.at[wid])

    @pl.when(jnp.logical_not(cont))
    def _last(nf=nf):
        for j in range(K):
            fb[nf, pl.ds(j * DD, 16)] = acc[j]
        plsc.store_scatter(fidx, [jnp.full((16,), nf, _i32)],
                           jnp.full((16,), r_cur, _i32), mask=iota == 0)

    nf = nf + jnp.logical_not(cont).astype(_i32)

    def pad(i, carry):
        @pl.when(i >= nf)
        def _():
            plsc.store_scatter(fidx, [jnp.full((16,), i, _i32)],
                               jnp.full((16,), DUMP, _i32), mask=iota == 0)
        return carry

    lax.fori_loop(0, FB, pad, 0)
    pltpu.sync_copy(fb, vout_hbm.at[fidx])
    pltpu.sync_copy(s_acc, sout_hbm.at[wid])


_SC_OUT = (
    jax.ShapeDtypeStruct((VROWS, D), _f32),
    jax.ShapeDtypeStruct((NW, NK), _f32),
    jax.ShapeDtypeStruct((NW, 1, D), _f32),
    jax.ShapeDtypeStruct((NW, 16), _i32),
)

_SC_SCRATCH = [
    pltpu.VMEM((NK,), _f32),
    pltpu.VMEM((CH,), _i32),
    pltpu.VMEM((CH,), _i32),
    pltpu.VMEM((CH,), _f32),
    pltpu.VMEM((CH, D), _f32),
    pltpu.VMEM((CH, D), _f32),
    pltpu.VMEM((FB, D), _f32),
    pltpu.VMEM((FB,), _i32),
    pltpu.VMEM((16,), _i32),
    pltpu.VMEM((1, D), _f32),
    pltpu.SemaphoreType.DMA,
    pltpu.SemaphoreType.DMA,
]


@functools.partial(pl.kernel, out_type=_SC_OUT, mesh=_MESH,
                   compiler_params=_SC_PARAMS, scratch_types=_SC_SCRATCH)
def _sc_iter(x_hbm, u_hbm, ppr_hbm, row_hbm, col_hbm,
             vout_hbm, sout_hbm, bnd_hbm, bndr_hbm,
             s_acc, row_v, col_v, ppr_v, xc_v, ur_v, fb, fidx,
             tmp_v, bnd_v, gsem, usem):
    _sc_pass_body(True, x_hbm, u_hbm, ppr_hbm, row_hbm, col_hbm,
                  vout_hbm, sout_hbm, bnd_hbm, bndr_hbm,
                  s_acc, row_v, col_v, ppr_v, xc_v, ur_v, fb, fidx,
                  tmp_v, bnd_v, gsem, usem)


@functools.partial(pl.kernel, out_type=_SC_OUT, mesh=_MESH,
                   compiler_params=_SC_PARAMS, scratch_types=_SC_SCRATCH)
def _sc_init(x_hbm, u_hbm, ppr_hbm, row_hbm, col_hbm,
             vout_hbm, sout_hbm, bnd_hbm, bndr_hbm,
             s_acc, row_v, col_v, ppr_v, xc_v, ur_v, fb, fidx,
             tmp_v, bnd_v, gsem, usem):
    _sc_pass_body(False, x_hbm, u_hbm, ppr_hbm, row_hbm, col_hbm,
                  vout_hbm, sout_hbm, bnd_hbm, bndr_hbm,
                  s_acc, row_v, col_v, ppr_v, xc_v, ur_v, fb, fidx,
                  tmp_v, bnd_v, gsem, usem)


@functools.partial(
    pl.kernel,
    out_type=jax.ShapeDtypeStruct((E,), _f32),
    mesh=_MESH,
    compiler_params=_SC_PARAMS,
    scratch_types=[
        pltpu.VMEM((N,), _f32),
        pltpu.VMEM((CH,), _i32),
        pltpu.VMEM((CH,), _f32),
        pltpu.VMEM((CH,), _f32),
    ],
)
def _sc_pprs(ppr_hbm, row_hbm, s0_hbm, out_hbm, s0_v, row_v, ppr_v, o_v):
    wid = lax.axis_index("s") * NC + lax.axis_index("c")
    pltpu.sync_copy(s0_hbm, s0_v)

    def chunk(i, carry):
        base = wid * EPW + i * CH
        pltpu.sync_copy(row_hbm.at[pl.ds(base, CH)], row_v)
        pltpu.sync_copy(ppr_hbm.at[pl.ds(base, CH)], ppr_v)
        for g in range(CH // 16):
            rid = row_v[pl.ds(g * 16, 16)]
            sv = plsc.load_gather(s0_v, [rid])
            o_v[pl.ds(g * 16, 16)] = (
                jnp.exp(ppr_v[pl.ds(g * 16, 16)]) / jnp.maximum(sv, 1e-30))
        pltpu.sync_copy(o_v, out_hbm.at[pl.ds(base, CH)])
        return carry

    lax.fori_loop(0, NCHUNK, chunk, 0)


def kernel(x_nb, ppr, row_idx, col_idx, x_idx, W_pca, b_pca, W_mlp, b_mlp):
    del x_idx
    row_idx = row_idx.astype(_i32)
    col_idx = col_idx.astype(_i32)
    cap = jnp.asarray(_CAPSUM)
    exp8 = jnp.asarray(_EXPAND8)

    x = _prep(x_nb, W_pca, b_pca.reshape(1, D), cap)
    v, sout, bnd, bndr = _sc_init(x, x, ppr, row_idx, col_idx)
    u, s0 = _fixup(v, sout.reshape(NW, N, K), bnd.reshape(NW, D), bndr,
                   exp8, cap, norm=False)
    pprs = _sc_pprs(ppr, row_idx, s0[:, 0])
    for it in range(3):
        v, sout, bnd, bndr = _sc_iter(x, u, pprs, row_idx, col_idx)
        if it < 2:
            u, _ = _fixup(v, sout.reshape(NW, N, K), bnd.reshape(NW, D),
                          bndr, exp8, cap, norm=True)
    return _final(v, sout.reshape(NW, N, K), bnd.reshape(NW, D), bndr,
                  exp8, W_mlp, b_mlp.reshape(1, NCLASS))


# speedup vs baseline: 1.8028x; 1.1013x over previous
import functools

import jax
import jax.numpy as jnp
import numpy as np
from jax import lax
from jax.experimental import pallas as pl
from jax.experimental.pallas import tpu as pltpu
from jax.experimental.pallas import tpu_sc as plsc

N = 10000
E = 320000
NFEAT = 128
D = 128
K = 8
DD = 16
NCLASS = 64

NC = 2
NS = 16
NW = NC * NS
EPW = E // NW
CH = 80
NCHUNK = EPW // CH
NK = N * K

_f32 = jnp.float32
_i32 = jnp.int32

_CAPSUM = np.zeros((D, D), np.float32)
for _k in range(K):
    _CAPSUM[_k * DD:(_k + 1) * DD, _k * DD:(_k + 1) * DD] = 1.0
_EXPAND8 = np.zeros((K, D), np.float32)
for _k in range(K):
    _EXPAND8[_k, _k * DD:(_k + 1) * DD] = 1.0

RB = 1000


def _prep_body(xnb_ref, w_ref, b_ref, cap_ref, o_ref):
    h = jnp.maximum(
        jax.lax.dot_general(xnb_ref[...], w_ref[...], (((1,), (0,)), ((), ())),
                            precision=lax.Precision.HIGHEST,
                            preferred_element_type=_f32) + b_ref[...], 0.0)
    q = jax.lax.dot_general(h * h, cap_ref[...], (((1,), (0,)), ((), ())),
                            precision=lax.Precision.HIGHEST,
                            preferred_element_type=_f32)
    o_ref[...] = h / jnp.maximum(jnp.sqrt(q), 1e-12)


def _prep(x_nb, W_pca, b_pca, cap):
    return pl.pallas_call(
        _prep_body,
        grid=(N // RB,),
        in_specs=[
            pl.BlockSpec((RB, NFEAT), lambda i: (i, 0)),
            pl.BlockSpec((NFEAT, D), lambda i: (0, 0)),
            pl.BlockSpec((1, D), lambda i: (0, 0)),
            pl.BlockSpec((D, D), lambda i: (0, 0)),
        ],
        out_specs=pl.BlockSpec((RB, D), lambda i: (i, 0)),
        out_shape=jax.ShapeDtypeStruct((N, D), _f32),
    )(x_nb, W_pca, b_pca, cap)


def _combine(i, v_ref, bnd_ref, bndr_ref):
    rows = i * RB + jax.lax.broadcasted_iota(_i32, (RB, NW), 0)
    oh = (rows == bndr_ref[...][:, 0][None, :]).astype(_f32)
    return v_ref[...] + jax.lax.dot_general(
        oh, bnd_ref[...], (((1,), (0,)), ((), ())),
        precision=lax.Precision.HIGHEST, preferred_element_type=_f32)


def _fixup_body(norm, v_ref, sp_ref, bnd_ref, bndr_ref, exp_ref, cap_ref,
                u_ref, s_ref):
    i = pl.program_id(0)
    feat = _combine(i, v_ref, bnd_ref, bndr_ref)
    s8 = jnp.sum(sp_ref[...], axis=0)
    sfull = jax.lax.dot_general(s8, exp_ref[...], (((1,), (0,)), ((), ())),
                                precision=lax.Precision.HIGHEST,
                                preferred_element_type=_f32)
    u = jnp.where(sfull > 0.0, feat / jnp.maximum(sfull, 1e-30), 0.0)
    if norm:
        q = jax.lax.dot_general(u * u, cap_ref[...], (((1,), (0,)), ((), ())),
                                precision=lax.Precision.HIGHEST,
                                preferred_element_type=_f32)
        u = u / jnp.maximum(jnp.sqrt(q), 1e-12)
    u_ref[...] = u
    s_ref[...] = s8


def _fixup(v, sp, bnd, bndr, exp8, cap, norm):
    return pl.pallas_call(
        functools.partial(_fixup_body, norm),
        grid=(N // RB,),
        in_specs=[
            pl.BlockSpec((RB, D), lambda i: (i, 0)),
            pl.BlockSpec((NW, RB, K), lambda i: (0, i, 0)),
            pl.BlockSpec((NW, D), lambda i: (0, 0)),
            pl.BlockSpec((NW, 16), lambda i: (0, 0)),
            pl.BlockSpec((K, D), lambda i: (0, 0)),
            pl.BlockSpec((D, D), lambda i: (0, 0)),
        ],
        out_specs=[
            pl.BlockSpec((RB, D), lambda i: (i, 0)),
            pl.BlockSpec((RB, K), lambda i: (i, 0)),
        ],
        out_shape=[
            jax.ShapeDtypeStruct((N, D), _f32),
            jax.ShapeDtypeStruct((N, K), _f32),
        ],
    )(v, sp, bnd, bndr, exp8, cap)


def _final_body(v_ref, sp_ref, bnd_ref, bndr_ref, exp_ref, wm_ref, bm_ref,
                o_ref):
    i = pl.program_id(0)
    feat = _combine(i, v_ref, bnd_ref, bndr_ref)
    s8 = jnp.sum(sp_ref[...], axis=0)
    sfull = jax.lax.dot_general(s8, exp_ref[...], (((1,), (0,)), ((), ())),
                                precision=lax.Precision.HIGHEST,
                                preferred_element_type=_f32)
    u = jnp.where(sfull > 0.0, feat / jnp.maximum(sfull, 1e-30), 0.0)
    h = jnp.maximum(u, 0.0)
    logits = jax.lax.dot_general(h, wm_ref[...], (((1,), (0,)), ((), ())),
                                 precision=lax.Precision.HIGHEST,
                                 preferred_element_type=_f32) + bm_ref[...]
    m = jnp.max(logits, axis=1, keepdims=True)
    lse = jnp.log(jnp.sum(jnp.exp(logits - m), axis=1, keepdims=True))
    o_ref[...] = logits - m - lse


def _final(v, sp, bnd, bndr, exp8, W_mlp, b_mlp):
    return pl.pallas_call(
        _final_body,
        grid=(N // RB,),
        in_specs=[
            pl.BlockSpec((RB, D), lambda i: (i, 0)),
            pl.BlockSpec((NW, RB, K), lambda i: (0, i, 0)),
            pl.BlockSpec((NW, D), lambda i: (0, 0)),
            pl.BlockSpec((NW, 16), lambda i: (0, 0)),
            pl.BlockSpec((K, D), lambda i: (0, 0)),
            pl.BlockSpec((D, NCLASS), lambda i: (0, 0)),
            pl.BlockSpec((1, NCLASS), lambda i: (0, 0)),
        ],
        out_specs=pl.BlockSpec((RB, NCLASS), lambda i: (i, 0)),
        out_shape=jax.ShapeDtypeStruct((N, NCLASS), _f32),
    )(v, sp, bnd, bndr, exp8, W_mlp, b_mlp)


_MESH = plsc.VectorSubcoreMesh(core_axis_name="c", subcore_axis_name="s")
_SC_PARAMS = pltpu.CompilerParams(needs_layout_passes=False)

FB = 80
VROWS = N + 16
DUMP = N


def _sc_pass_body(has_u, x_hbm, u_hbm, ppr_hbm, row_hbm, col_hbm,
                  vout_hbm, sout_hbm, bnd_hbm, bndr_hbm,
                  s_acc, row_v, col_v, ppr_v, xc_v, ur_v, fb, fidx,
                  tmp_v, bnd_v, gsem, usem, isem):
    c = lax.axis_index("c")
    sid = lax.axis_index("s")
    wid = sid * NC + c
    iota = lax.iota(_i32, 16)
    z16 = jnp.zeros((16,), _f32)

    def zloop(i, carry):
        s_acc[pl.ds(i * 16, 16)] = z16
        return carry

    lax.fori_loop(0, NK // 16, zloop, 0)
    for g in range(FB // 16):
        fidx[pl.ds(g * 16, 16)] = jnp.full((16,), DUMP, _i32)

    pltpu.sync_copy(row_hbm.at[pl.ds(wid * EPW, 16)], tmp_v)
    r_first = tmp_v[...][0]

    @pl.when(wid < NW - 1)
    def _probe():
        pltpu.sync_copy(row_hbm.at[pl.ds((wid + 1) * EPW, 16)], tmp_v)

    nx = tmp_v[...][0]
    r_next = jnp.where(wid < NW - 1, nx, -1)

    acc0 = tuple(z16 for _ in range(K))

    def chunk(i, carry):
        base = wid * EPW + i * CH
        c1 = pltpu.async_copy(row_hbm.at[pl.ds(base, CH)], row_v, isem)
        c2 = pltpu.async_copy(col_hbm.at[pl.ds(base, CH)], col_v, isem)
        c3 = pltpu.async_copy(ppr_hbm.at[pl.ds(base, CH)], ppr_v, isem)
        c1.wait()
        c2.wait()
        c3.wait()
        cx = pltpu.async_copy(x_hbm.at[col_v], xc_v, gsem)
        if has_u:
            cu = pltpu.async_copy(u_hbm.at[row_v], ur_v, usem)
        cx.wait()
        if has_u:
            cu.wait()

        def group(g, gcarry):
            r_cur, nf, acc = gcarry

            @pl.when(nf + 16 > FB)
            def _scat():
                pltpu.sync_copy(fb, vout_hbm.at[fidx])

            nf = jnp.where(nf + 16 > FB, 0, nf)
            pv = ppr_v[pl.ds(g * 16, 16)]
            rv = row_v[pl.ds(g * 16, 16)]
            if not has_u:
                wg = jnp.exp(pv)
            for l in range(16):
                e = g * 16 + l
                r = rv[l]
                xcs = [xc_v[e, pl.ds(j * DD, 16)] for j in range(K)]
                if has_u:
                    pe = pv[l]
                    tvec = jnp.zeros((16,), _f32)
                    for j in range(K):
                        tj = jnp.sum(xcs[j] * ur_v[e, pl.ds(j * DD, 16)])
                        tvec = jnp.where(iota == j, tj, tvec)
                    wv = jnp.where(iota < K, jnp.exp(pe * tvec), 0.0)
                else:
                    wv = jnp.where(iota < K, jnp.full((16,), wg[l], _f32),
                                   0.0)
                plsc.addupdate_scatter(s_acc, [r * K + iota], wv,
                                       mask=iota < K)
                flush = r != r_cur

                @pl.when(flush)
                def _flush(nf=nf, r_cur=r_cur, acc=acc):
                    for j in range(K):
                        fb[nf, pl.ds(j * DD, 16)] = acc[j]
                    plsc.store_scatter(fidx, [jnp.full((16,), nf, _i32)],
                                       jnp.full((16,), r_cur, _i32),
                                       mask=iota == 0)

                nf = nf + flush.astype(_i32)
                keep = jnp.where(flush, 0.0, 1.0)
                if has_u:
                    wjs = [jnp.full((16,), wv[j], _f32) for j in range(K)]
                else:
                    wjs = [wv] * K
                acc = tuple(acc[j] * keep + xcs[j] * wjs[j]
                            for j in range(K))
                r_cur = r
            return (r_cur, nf, acc)

        return lax.fori_loop(0, CH // 16, group, carry)

    r_cur, nf, acc = lax.fori_loop(0, NCHUNK, chunk,
                                   (r_first, jnp.int32(0), acc0))

    for j in range(K):
        bnd_v[0, pl.ds(j * DD, 16)] = acc[j]
    cont = r_cur == r_next
    bndrow = jnp.where(cont, r_cur, DUMP)
    tmp_v[...] = jnp.full((16,), bndrow, _i32)
    pltpu.sync_copy(bnd_v, bnd_hbm.at[wid])
    pltpu.sync_copy(tmp_v, bndr_hbm.at[wid])

    @pl.when(jnp.logical_not(cont))
    def _last(nf=nf):
        for j in range(K):
            fb[nf, pl.ds(j * DD, 16)] = acc[j]
        plsc.store_scatter(fidx, [jnp.full((16,), nf, _i32)],
                           jnp.full((16,), r_cur, _i32), mask=iota == 0)

    nf = nf + jnp.logical_not(cont).astype(_i32)

    def pad(i, carry):
        @pl.when(i >= nf)
        def _():
            plsc.store_scatter(fidx, [jnp.full((16,), i, _i32)],
                               jnp.full((16,), DUMP, _i32), mask=iota == 0)
        return carry

    lax.fori_loop(0, FB, pad, 0)
    pltpu.sync_copy(fb, vout_hbm.at[fidx])
    pltpu.sync_copy(s_acc, sout_hbm.at[wid])


_SC_OUT = (
    jax.ShapeDtypeStruct((VROWS, D), _f32),
    jax.ShapeDtypeStruct((NW, NK), _f32),
    jax.ShapeDtypeStruct((NW, 1, D), _f32),
    jax.ShapeDtypeStruct((NW, 16), _i32),
)

_SC_SCRATCH = [
    pltpu.VMEM((NK,), _f32),
    pltpu.VMEM((CH,), _i32),
    pltpu.VMEM((CH,), _i32),
    pltpu.VMEM((CH,), _f32),
    pltpu.VMEM((CH, D), _f32),
    pltpu.VMEM((CH, D), _f32),
    pltpu.VMEM((FB, D), _f32),
    pltpu.VMEM((FB,), _i32),
    pltpu.VMEM((16,), _i32),
    pltpu.VMEM((1, D), _f32),
    pltpu.SemaphoreType.DMA,
    pltpu.SemaphoreType.DMA,
    pltpu.SemaphoreType.DMA,
]


@functools.partial(pl.kernel, out_type=_SC_OUT, mesh=_MESH,
                   compiler_params=_SC_PARAMS, scratch_types=_SC_SCRATCH)
def _sc_iter(x_hbm, u_hbm, ppr_hbm, row_hbm, col_hbm,
             vout_hbm, sout_hbm, bnd_hbm, bndr_hbm,
             s_acc, row_v, col_v, ppr_v, xc_v, ur_v, fb, fidx,
             tmp_v, bnd_v, gsem, usem, isem):
    _sc_pass_body(True, x_hbm, u_hbm, ppr_hbm, row_hbm, col_hbm,
                  vout_hbm, sout_hbm, bnd_hbm, bndr_hbm,
                  s_acc, row_v, col_v, ppr_v, xc_v, ur_v, fb, fidx,
                  tmp_v, bnd_v, gsem, usem, isem)


@functools.partial(pl.kernel, out_type=_SC_OUT, mesh=_MESH,
                   compiler_params=_SC_PARAMS, scratch_types=_SC_SCRATCH)
def _sc_init(x_hbm, u_hbm, ppr_hbm, row_hbm, col_hbm,
             vout_hbm, sout_hbm, bnd_hbm, bndr_hbm,
             s_acc, row_v, col_v, ppr_v, xc_v, ur_v, fb, fidx,
             tmp_v, bnd_v, gsem, usem, isem):
    _sc_pass_body(False, x_hbm, u_hbm, ppr_hbm, row_hbm, col_hbm,
                  vout_hbm, sout_hbm, bnd_hbm, bndr_hbm,
                  s_acc, row_v, col_v, ppr_v, xc_v, ur_v, fb, fidx,
                  tmp_v, bnd_v, gsem, usem, isem)


@functools.partial(
    pl.kernel,
    out_type=jax.ShapeDtypeStruct((E,), _f32),
    mesh=_MESH,
    compiler_params=_SC_PARAMS,
    scratch_types=[
        pltpu.VMEM((N,), _f32),
        pltpu.VMEM((CH,), _i32),
        pltpu.VMEM((CH,), _f32),
        pltpu.VMEM((CH,), _f32),
    ],
)
def _sc_pprs(ppr_hbm, row_hbm, s0_hbm, out_hbm, s0_v, row_v, ppr_v, o_v):
    wid = lax.axis_index("s") * NC + lax.axis_index("c")
    pltpu.sync_copy(s0_hbm, s0_v)

    def chunk(i, carry):
        base = wid * EPW + i * CH
        pltpu.sync_copy(row_hbm.at[pl.ds(base, CH)], row_v)
        pltpu.sync_copy(ppr_hbm.at[pl.ds(base, CH)], ppr_v)
        for g in range(CH // 16):
            rid = row_v[pl.ds(g * 16, 16)]
            sv = plsc.load_gather(s0_v, [rid])
            o_v[pl.ds(g * 16, 16)] = (
                jnp.exp(ppr_v[pl.ds(g * 16, 16)]) / jnp.maximum(sv, 1e-30))
        pltpu.sync_copy(o_v, out_hbm.at[pl.ds(base, CH)])
        return carry

    lax.fori_loop(0, NCHUNK, chunk, 0)


def kernel(x_nb, ppr, row_idx, col_idx, x_idx, W_pca, b_pca, W_mlp, b_mlp):
    del x_idx
    row_idx = row_idx.astype(_i32)
    col_idx = col_idx.astype(_i32)
    cap = jnp.asarray(_CAPSUM)
    exp8 = jnp.asarray(_EXPAND8)

    x = _prep(x_nb, W_pca, b_pca.reshape(1, D), cap)
    v, sout, bnd, bndr = _sc_init(x, x, ppr, row_idx, col_idx)
    u, s0 = _fixup(v, sout.reshape(NW, N, K), bnd.reshape(NW, D), bndr,
                   exp8, cap, norm=False)
    pprs = _sc_pprs(ppr, row_idx, s0[:, 0])
    for it in range(3):
        v, sout, bnd, bndr = _sc_iter(x, u, pprs, row_idx, col_idx)
        if it < 2:
            u, _ = _fixup(v, sout.reshape(NW, N, K), bnd.reshape(NW, D),
                          bndr, exp8, cap, norm=True)
    return _final(v, sout.reshape(NW, N, K), bnd.reshape(NW, D), bndr,
                  exp8, W_mlp, b_mlp.reshape(1, NCLASS))
